# Initial kernel scaffold; baseline (speedup 1.0000x reference)
import jax
import jax.numpy as jnp
from jax.experimental import pallas as pl


def kernel(var_f, con_f, combined_edge_f, edge_index_var_con, vW1, vb1, vW2, vb2, cW1, cb1, cW2, cb2, eW1, eb1, eW2, eb2):
    raise NotImplementedError("write your pallas kernel here")


# speedup vs baseline: 3.6273x; 3.6273x over previous
import functools

import jax
import jax.numpy as jnp
from jax import lax
from jax.experimental import pallas as pl
from jax.experimental.pallas import tpu as pltpu
from jax.experimental.pallas import tpu_sc as plsc

N_VAR, N_CON, E, D = 100000, 50000, 1600000, 16


def _node_body(x_ref, w1_ref, b1_ref, w2_ref, b2_ref, wp_ref, o_ref):
    h = jnp.maximum(jnp.dot(x_ref[...], w1_ref[...],
                            preferred_element_type=jnp.float32) + b1_ref[...], 0.0)
    h = jnp.maximum(jnp.dot(h, w2_ref[...],
                            preferred_element_type=jnp.float32) + b2_ref[...], 0.0)
    o_ref[...] = jnp.dot(h, wp_ref[...], preferred_element_type=jnp.float32)


def _node(x, w1, b1, w2, b2, wp, block):
    n = x.shape[0]
    block = min(block, n)
    wspec = pl.BlockSpec((16, 16), lambda i: (0, 0))
    bspec = pl.BlockSpec((1, 16), lambda i: (0, 0))
    return pl.pallas_call(
        _node_body,
        grid=(n // block,),
        in_specs=[pl.BlockSpec((block, 16), lambda i: (i, 0)),
                  wspec, bspec, wspec, bspec, wspec],
        out_specs=pl.BlockSpec((block, 16), lambda i: (i, 0)),
        out_shape=jax.ShapeDtypeStruct((n, 16), jnp.float32),
    )(x, w1, b1.reshape(1, 16), w2, b2.reshape(1, 16), wp)


_CHUNK = 2000


def _gather_add(var_p, con_p, idx0, idx1):
    info = plsc.get_sparse_core_info()
    nc, ns = info.num_cores, info.num_subcores
    nw = nc * ns
    epw = E // nw
    nchunk = epw // _CHUNK

    mesh = plsc.VectorSubcoreMesh(core_axis_name="c", subcore_axis_name="s")

    @functools.partial(
        pl.kernel,
        out_type=jax.ShapeDtypeStruct((E, 16), jnp.float32),
        mesh=mesh,
        compiler_params=pltpu.CompilerParams(use_tc_tiling_on_sc=False),
        scratch_types=[
            pltpu.VMEM((_CHUNK,), jnp.int32),
            pltpu.VMEM((_CHUNK,), jnp.int32),
            pltpu.VMEM((_CHUNK, 16), jnp.float32),
            pltpu.VMEM((_CHUNK, 16), jnp.float32),
            pltpu.SemaphoreType.DMA,
        ],
    )
    def gk(varp_hbm, conp_hbm, idx0_hbm, idx1_hbm, out_hbm,
           idx0_v, idx1_v, vrows, crows, sem):
        wid = lax.axis_index("s") * nc + lax.axis_index("c")

        def chunk(ci, carry):
            base = wid * epw + ci * _CHUNK
            pltpu.sync_copy(idx0_hbm.at[pl.ds(base, _CHUNK)], idx0_v)
            pltpu.sync_copy(idx1_hbm.at[pl.ds(base, _CHUNK)], idx1_v)
            pltpu.async_copy(varp_hbm.at[idx0_v], vrows, sem).wait()
            pltpu.async_copy(conp_hbm.at[idx1_v], crows, sem).wait()

            def addrow(r, c2):
                vrows[r, :] = vrows[r, :] + crows[r, :]
                return c2
            lax.fori_loop(0, _CHUNK, addrow, 0, unroll=4)
            pltpu.sync_copy(vrows, out_hbm.at[pl.ds(base, _CHUNK), :])
            return carry

        lax.fori_loop(0, nchunk, chunk, 0)

    return gk(var_p, con_p, idx0, idx1)


def _edge_body(ce_ref, g_ref, w1_ref, b1_ref, w2_ref, b2_ref, o_ref):
    pre = jnp.dot(ce_ref[...], w1_ref[...],
                  preferred_element_type=jnp.float32) + g_ref[...] + b1_ref[...]
    h = jnp.maximum(pre, 0.0)
    o_ref[...] = jnp.dot(h, w2_ref[...],
                         preferred_element_type=jnp.float32) + b2_ref[...]


def _edge(ce, g, w1e, b1, w2, b2, block=6400):
    wspec = pl.BlockSpec((16, 16), lambda i: (0, 0))
    bspec = pl.BlockSpec((1, 16), lambda i: (0, 0))
    return pl.pallas_call(
        _edge_body,
        grid=(E // block,),
        in_specs=[pl.BlockSpec((block, 16), lambda i: (i, 0)),
                  pl.BlockSpec((block, 16), lambda i: (i, 0)),
                  wspec, bspec, wspec, bspec],
        out_specs=pl.BlockSpec((block, 16), lambda i: (i, 0)),
        out_shape=jax.ShapeDtypeStruct((E, 16), jnp.float32),
    )(ce, g, w1e, b1.reshape(1, 16), w2, b2.reshape(1, 16))


def kernel(var_f, con_f, combined_edge_f, edge_index_var_con,
           vW1, vb1, vW2, vb2, cW1, cb1, cW2, cb2, eW1, eb1, eW2, eb2):
    idx0 = edge_index_var_con[0]
    idx1 = edge_index_var_con[1]
    eW1_e, eW1_v, eW1_c = eW1[:16], eW1[16:32], eW1[32:48]
    var_p = _node(var_f, vW1, vb1, vW2, vb2, eW1_v, block=4000)
    con_p = _node(con_f, cW1, cb1, cW2, cb2, eW1_c, block=2000)
    g = _gather_add(var_p, con_p, idx0, idx1)
    return _edge(combined_edge_f, g, eW1_e, eb1, eW2, eb2)


# speedup vs baseline: 6.0602x; 1.6707x over previous
import functools

import jax
import jax.numpy as jnp
from jax import lax
from jax.experimental import pallas as pl
from jax.experimental.pallas import tpu as pltpu
from jax.experimental.pallas import tpu_sc as plsc

N_VAR, N_CON, E, D = 100000, 50000, 1600000, 16


def _node_body(x_ref, w1_ref, b1_ref, w2_ref, b2_ref, wp_ref, o_ref):
    h = jnp.maximum(jnp.dot(w1_ref[...], x_ref[...],
                            preferred_element_type=jnp.float32) + b1_ref[...], 0.0)
    h = jnp.maximum(jnp.dot(w2_ref[...], h,
                            preferred_element_type=jnp.float32) + b2_ref[...], 0.0)
    o_ref[...] = jnp.dot(wp_ref[...], h, preferred_element_type=jnp.float32)


def _node(xT, w1, b1, w2, b2, wp, block):
    n = xT.shape[1]
    block = min(block, n)
    wspec = pl.BlockSpec((16, 16), lambda i: (0, 0))
    bspec = pl.BlockSpec((16, 1), lambda i: (0, 0))
    return pl.pallas_call(
        _node_body,
        grid=(n // block,),
        in_specs=[pl.BlockSpec((16, block), lambda i: (0, i)),
                  wspec, bspec, wspec, bspec, wspec],
        out_specs=pl.BlockSpec((16, block), lambda i: (0, i)),
        out_shape=jax.ShapeDtypeStruct((16, n), jnp.float32),
    )(xT, w1.T, b1.reshape(16, 1), w2.T, b2.reshape(16, 1), wp.T)


_CHUNK = 2000


def _gather_add(var_p, con_p, idx0, idx1):
    info = plsc.get_sparse_core_info()
    nc, ns = info.num_cores, info.num_subcores
    nw = nc * ns
    epw = E // nw
    nchunk = epw // _CHUNK

    mesh = plsc.VectorSubcoreMesh(core_axis_name="c", subcore_axis_name="s")

    @functools.partial(
        pl.kernel,
        out_type=jax.ShapeDtypeStruct((E, 16), jnp.float32),
        mesh=mesh,
        compiler_params=pltpu.CompilerParams(use_tc_tiling_on_sc=False),
        scratch_types=[
            pltpu.VMEM((_CHUNK,), jnp.int32),
            pltpu.VMEM((_CHUNK,), jnp.int32),
            pltpu.VMEM((_CHUNK, 16), jnp.float32),
            pltpu.VMEM((_CHUNK, 16), jnp.float32),
            pltpu.SemaphoreType.DMA,
        ],
    )
    def gk(varp_hbm, conp_hbm, idx0_hbm, idx1_hbm, out_hbm,
           idx0_v, idx1_v, vrows, crows, sem):
        wid = lax.axis_index("s") * nc + lax.axis_index("c")

        def chunk(ci, carry):
            base = wid * epw + ci * _CHUNK
            pltpu.sync_copy(idx0_hbm.at[pl.ds(base, _CHUNK)], idx0_v)
            pltpu.sync_copy(idx1_hbm.at[pl.ds(base, _CHUNK)], idx1_v)
            pltpu.async_copy(varp_hbm.at[idx0_v], vrows, sem).wait()
            pltpu.async_copy(conp_hbm.at[idx1_v], crows, sem).wait()

            def addrow(r, c2):
                vrows[r, :] = vrows[r, :] + crows[r, :]
                return c2
            lax.fori_loop(0, _CHUNK, addrow, 0, unroll=8)
            pltpu.sync_copy(vrows, out_hbm.at[pl.ds(base, _CHUNK), :])
            return carry

        lax.fori_loop(0, nchunk, chunk, 0)

    return gk(var_p, con_p, idx0, idx1)


def _edge_body(ceT_ref, gT_ref, w1_ref, b1_ref, w2_ref, b2_ref, o_ref):
    pre = jnp.dot(w1_ref[...], ceT_ref[...],
                  preferred_element_type=jnp.float32) + gT_ref[...] + b1_ref[...]
    h = jnp.maximum(pre, 0.0)
    o_ref[...] = jnp.dot(w2_ref[...], h,
                         preferred_element_type=jnp.float32) + b2_ref[...]


def _edge(ceT, gT, w1e, b1, w2, b2, block=64000):
    block = min(block, E)
    wspec = pl.BlockSpec((16, 16), lambda i: (0, 0))
    bspec = pl.BlockSpec((16, 1), lambda i: (0, 0))
    return pl.pallas_call(
        _edge_body,
        grid=(E // block,),
        in_specs=[pl.BlockSpec((16, block), lambda i: (0, i)),
                  pl.BlockSpec((16, block), lambda i: (0, i)),
                  wspec, bspec, wspec, bspec],
        out_specs=pl.BlockSpec((16, block), lambda i: (0, i)),
        out_shape=jax.ShapeDtypeStruct((16, E), jnp.float32),
    )(ceT, gT, w1e.T, b1.reshape(16, 1), w2.T, b2.reshape(16, 1))


def kernel(var_f, con_f, combined_edge_f, edge_index_var_con,
           vW1, vb1, vW2, vb2, cW1, cb1, cW2, cb2, eW1, eb1, eW2, eb2):
    idx0 = edge_index_var_con[0]
    idx1 = edge_index_var_con[1]
    eW1_e, eW1_v, eW1_c = eW1[:16], eW1[16:32], eW1[32:48]
    var_pT = _node(var_f.T, vW1, vb1, vW2, vb2, eW1_v, block=N_VAR)
    con_pT = _node(con_f.T, cW1, cb1, cW2, cb2, eW1_c, block=N_CON)
    g = _gather_add(var_pT.T, con_pT.T, idx0, idx1)
    outT = _edge(combined_edge_f.T, g.T, eW1_e, eb1, eW2, eb2)
    return outT.T


# speedup vs baseline: 6.9326x; 1.1439x over previous
import functools

import jax
import jax.numpy as jnp
from jax import lax
from jax.experimental import pallas as pl
from jax.experimental.pallas import tpu as pltpu
from jax.experimental.pallas import tpu_sc as plsc

N_VAR, N_CON, E, D = 100000, 50000, 1600000, 16


def _node_body(x_ref, w1_ref, b1_ref, w2_ref, b2_ref, wp_ref, o_ref):
    h = jnp.maximum(jnp.dot(w1_ref[...], x_ref[...],
                            preferred_element_type=jnp.float32) + b1_ref[...], 0.0)
    h = jnp.maximum(jnp.dot(w2_ref[...], h,
                            preferred_element_type=jnp.float32) + b2_ref[...], 0.0)
    o_ref[...] = jnp.dot(wp_ref[...], h, preferred_element_type=jnp.float32)


def _node(xT, w1, b1, w2, b2, wp, block):
    n = xT.shape[1]
    block = min(block, n)
    wspec = pl.BlockSpec((16, 16), lambda i: (0, 0))
    bspec = pl.BlockSpec((16, 1), lambda i: (0, 0))
    return pl.pallas_call(
        _node_body,
        grid=(n // block,),
        in_specs=[pl.BlockSpec((16, block), lambda i: (0, i)),
                  wspec, bspec, wspec, bspec, wspec],
        out_specs=pl.BlockSpec((16, block), lambda i: (0, i)),
        out_shape=jax.ShapeDtypeStruct((16, n), jnp.float32),
    )(xT, w1.T, b1.reshape(16, 1), w2.T, b2.reshape(16, 1), wp.T)


_CHUNK = 1000


def _gather_add(var_p, con_p, idx0, idx1):
    info = plsc.get_sparse_core_info()
    nc, ns = info.num_cores, info.num_subcores
    nw = nc * ns
    epw = E // nw
    nchunk = epw // _CHUNK
    C = _CHUNK

    mesh = plsc.VectorSubcoreMesh(core_axis_name="c", subcore_axis_name="s")

    @functools.partial(
        pl.kernel,
        out_type=jax.ShapeDtypeStruct((E, 16), jnp.float32),
        mesh=mesh,
        compiler_params=pltpu.CompilerParams(use_tc_tiling_on_sc=False),
        scratch_types=[
            pltpu.VMEM((C,), jnp.int32), pltpu.VMEM((C,), jnp.int32),
            pltpu.VMEM((C,), jnp.int32), pltpu.VMEM((C,), jnp.int32),
            pltpu.VMEM((C, 16), jnp.float32), pltpu.VMEM((C, 16), jnp.float32),
            pltpu.VMEM((C, 16), jnp.float32), pltpu.VMEM((C, 16), jnp.float32),
            pltpu.SemaphoreType.DMA, pltpu.SemaphoreType.DMA,
            pltpu.SemaphoreType.DMA, pltpu.SemaphoreType.DMA,
            pltpu.SemaphoreType.DMA, pltpu.SemaphoreType.DMA,
        ],
    )
    def gk(varp_hbm, conp_hbm, idx0_hbm, idx1_hbm, out_hbm,
           i0a, i0b, i1a, i1b, va, vb, ca, cb,
           gsa, gsb, isa, isb, osa, osb):
        wid = lax.axis_index("s") * nc + lax.axis_index("c")
        wbase = wid * epw
        slot_a = (i0a, i1a, va, ca, gsa, isa, osa)
        slot_b = (i0b, i1b, vb, cb, gsb, isb, osb)

        def idx_start(e, s):
            (i0s, i1s, _, _, _, iss, _) = s
            pltpu.async_copy(idx0_hbm.at[pl.ds(wbase + e * C, C)], i0s, iss)
            pltpu.async_copy(idx1_hbm.at[pl.ds(wbase + e * C, C)], i1s, iss)

        def idx_wait(s):
            (i0s, i1s, _, _, _, iss, _) = s
            pltpu.make_async_copy(idx0_hbm.at[pl.ds(wbase, C)], i0s, iss).wait()
            pltpu.make_async_copy(idx1_hbm.at[pl.ds(wbase, C)], i1s, iss).wait()

        def gather_start(s):
            (i0s, i1s, vs, cs, gss, _, _) = s
            pltpu.async_copy(varp_hbm.at[i0s], vs, gss)
            pltpu.async_copy(conp_hbm.at[i1s], cs, gss)

        def gather_wait(s):
            (i0s, i1s, vs, cs, gss, _, _) = s
            pltpu.make_async_copy(varp_hbm.at[i0s], vs, gss).wait()
            pltpu.make_async_copy(conp_hbm.at[i1s], cs, gss).wait()

        def out_wait(s):
            (_, _, vs, _, _, _, oss) = s
            pltpu.make_async_copy(vs, out_hbm.at[pl.ds(wbase, C), :], oss).wait()

        def half(e, s, n):
            (i0s, i1s, vs, cs, gss, iss, oss) = s
            gather_wait(s)

            @pl.when(e + 1 < nchunk)
            def _():
                idx_wait(n)

                @pl.when(e >= 1)
                def _():
                    out_wait(n)
                gather_start(n)

            @pl.when(e + 2 < nchunk)
            def _():
                idx_start(e + 2, s)

            def addrow(r, c2):
                vs[r, :] = vs[r, :] + cs[r, :]
                return c2
            lax.fori_loop(0, C, addrow, 0, unroll=8)
            pltpu.async_copy(vs, out_hbm.at[pl.ds(wbase + e * C, C), :], oss)

        (i0s, i1s, _, _, _, _, _) = slot_a
        pltpu.sync_copy(idx0_hbm.at[pl.ds(wbase, C)], i0s)
        pltpu.sync_copy(idx1_hbm.at[pl.ds(wbase, C)], i1s)
        gather_start(slot_a)
        idx_start(1, slot_b)

        def pair(k, carry):
            half(2 * k, slot_a, slot_b)
            half(2 * k + 1, slot_b, slot_a)
            return carry

        lax.fori_loop(0, nchunk // 2, pair, 0)
        out_wait(slot_a)
        out_wait(slot_b)

    return gk(var_p, con_p, idx0, idx1)


def _edge_body(ceT_ref, gT_ref, w1_ref, b1_ref, w2_ref, b2_ref, o_ref):
    pre = jnp.dot(w1_ref[...], ceT_ref[...],
                  preferred_element_type=jnp.float32) + gT_ref[...] + b1_ref[...]
    h = jnp.maximum(pre, 0.0)
    o_ref[...] = jnp.dot(w2_ref[...], h,
                         preferred_element_type=jnp.float32) + b2_ref[...]


def _edge(ceT, gT, w1e, b1, w2, b2, block=64000):
    block = min(block, E)
    wspec = pl.BlockSpec((16, 16), lambda i: (0, 0))
    bspec = pl.BlockSpec((16, 1), lambda i: (0, 0))
    return pl.pallas_call(
        _edge_body,
        grid=(E // block,),
        in_specs=[pl.BlockSpec((16, block), lambda i: (0, i)),
                  pl.BlockSpec((16, block), lambda i: (0, i)),
                  wspec, bspec, wspec, bspec],
        out_specs=pl.BlockSpec((16, block), lambda i: (0, i)),
        out_shape=jax.ShapeDtypeStruct((16, E), jnp.float32),
    )(ceT, gT, w1e.T, b1.reshape(16, 1), w2.T, b2.reshape(16, 1))


def kernel(var_f, con_f, combined_edge_f, edge_index_var_con,
           vW1, vb1, vW2, vb2, cW1, cb1, cW2, cb2, eW1, eb1, eW2, eb2):
    idx0 = edge_index_var_con[0]
    idx1 = edge_index_var_con[1]
    eW1_e, eW1_v, eW1_c = eW1[:16], eW1[16:32], eW1[32:48]
    var_pT = _node(var_f.T, vW1, vb1, vW2, vb2, eW1_v, block=N_VAR)
    con_pT = _node(con_f.T, cW1, cb1, cW2, cb2, eW1_c, block=N_CON)
    g = _gather_add(var_pT.T, con_pT.T, idx0, idx1)
    outT = _edge(combined_edge_f.T, g.T, eW1_e, eb1, eW2, eb2)
    return outT.T


# speedup vs baseline: 6.9410x; 1.0012x over previous
import functools

import jax
import jax.numpy as jnp
from jax import lax
from jax.experimental import pallas as pl
from jax.experimental.pallas import tpu as pltpu
from jax.experimental.pallas import tpu_sc as plsc

N_VAR, N_CON, E, D = 100000, 50000, 1600000, 16


def _node_body(x_ref, w1_ref, b1_ref, w2_ref, b2_ref, wp_ref, o_ref):
    h = jnp.maximum(jnp.dot(w1_ref[...], x_ref[...],
                            preferred_element_type=jnp.float32) + b1_ref[...], 0.0)
    h = jnp.maximum(jnp.dot(w2_ref[...], h,
                            preferred_element_type=jnp.float32) + b2_ref[...], 0.0)
    o_ref[...] = jnp.dot(wp_ref[...], h, preferred_element_type=jnp.float32)


def _node(xT, w1, b1, w2, b2, wp, block):
    n = xT.shape[1]
    block = min(block, n)
    wspec = pl.BlockSpec((16, 16), lambda i: (0, 0))
    bspec = pl.BlockSpec((16, 1), lambda i: (0, 0))
    return pl.pallas_call(
        _node_body,
        grid=(n // block,),
        in_specs=[pl.BlockSpec((16, block), lambda i: (0, i)),
                  wspec, bspec, wspec, bspec, wspec],
        out_specs=pl.BlockSpec((16, block), lambda i: (0, i)),
        out_shape=jax.ShapeDtypeStruct((16, n), jnp.float32),
    )(xT, w1.T, b1.reshape(16, 1), w2.T, b2.reshape(16, 1), wp.T)


_CHUNK = 1000


def _gather_add(var_p, con_p, idx0, idx1):
    info = plsc.get_sparse_core_info()
    nc, ns = info.num_cores, info.num_subcores
    nw = nc * ns
    epw = E // nw
    nchunk = epw // _CHUNK
    C = _CHUNK

    mesh = plsc.VectorSubcoreMesh(core_axis_name="c", subcore_axis_name="s")

    @functools.partial(
        pl.kernel,
        out_type=jax.ShapeDtypeStruct((E // 8, 128), jnp.float32),
        mesh=mesh,
        compiler_params=pltpu.CompilerParams(use_tc_tiling_on_sc=False),
        scratch_types=[
            pltpu.VMEM((C,), jnp.int32), pltpu.VMEM((C,), jnp.int32),
            pltpu.VMEM((C,), jnp.int32), pltpu.VMEM((C,), jnp.int32),
            pltpu.VMEM((C, 16), jnp.float32), pltpu.VMEM((C, 16), jnp.float32),
            pltpu.VMEM((C, 16), jnp.float32), pltpu.VMEM((C, 16), jnp.float32),
            pltpu.VMEM((C // 8, 128), jnp.float32), pltpu.VMEM((C // 8, 128), jnp.float32),
            pltpu.SemaphoreType.DMA, pltpu.SemaphoreType.DMA,
            pltpu.SemaphoreType.DMA, pltpu.SemaphoreType.DMA,
            pltpu.SemaphoreType.DMA, pltpu.SemaphoreType.DMA,
        ],
    )
    def gk(varp_hbm, conp_hbm, idx0_hbm, idx1_hbm, out_hbm,
           i0a, i0b, i1a, i1b, va, vb, ca, cb, oba, obb,
           gsa, gsb, isa, isb, osa, osb):
        wid = lax.axis_index("s") * nc + lax.axis_index("c")
        wbase = wid * epw
        slot_a = (i0a, i1a, va, ca, oba, gsa, isa, osa)
        slot_b = (i0b, i1b, vb, cb, obb, gsb, isb, osb)

        def idx_start(e, s):
            (i0s, i1s, _, _, _, _, iss, _) = s
            pltpu.async_copy(idx0_hbm.at[pl.ds(wbase + e * C, C)], i0s, iss)
            pltpu.async_copy(idx1_hbm.at[pl.ds(wbase + e * C, C)], i1s, iss)

        def idx_wait(s):
            (i0s, i1s, _, _, _, _, iss, _) = s
            pltpu.make_async_copy(idx0_hbm.at[pl.ds(wbase, C)], i0s, iss).wait()
            pltpu.make_async_copy(idx1_hbm.at[pl.ds(wbase, C)], i1s, iss).wait()

        def gather_start(s):
            (i0s, i1s, vs, cs, _, gss, _, _) = s
            pltpu.async_copy(varp_hbm.at[i0s], vs, gss)
            pltpu.async_copy(conp_hbm.at[i1s], cs, gss)

        def gather_wait(s):
            (i0s, i1s, vs, cs, _, gss, _, _) = s
            pltpu.make_async_copy(varp_hbm.at[i0s], vs, gss).wait()
            pltpu.make_async_copy(conp_hbm.at[i1s], cs, gss).wait()

        def out_wait(s):
            (_, _, _, _, obs, _, _, oss) = s
            pltpu.make_async_copy(
                obs, out_hbm.at[pl.ds(wbase // 8, C // 8), :], oss).wait()

        def half(e, s, n):
            (i0s, i1s, vs, cs, obs, gss, iss, oss) = s
            gather_wait(s)

            @pl.when(e + 1 < nchunk)
            def _():
                idx_wait(n)
                gather_start(n)

            @pl.when(e + 2 < nchunk)
            def _():
                idx_start(e + 2, s)

            @pl.when(e >= 2)
            def _():
                out_wait(s)

            def addrow(r, c2):
                obs[r // 8, pl.ds((r % 8) * 16, 16)] = vs[r, :] + cs[r, :]
                return c2
            lax.fori_loop(0, C, addrow, 0, unroll=8)
            pltpu.async_copy(
                obs, out_hbm.at[pl.ds((wbase + e * C) // 8, C // 8), :], oss)

        (i0s, i1s, _, _, _, _, _, _) = slot_a
        pltpu.sync_copy(idx0_hbm.at[pl.ds(wbase, C)], i0s)
        pltpu.sync_copy(idx1_hbm.at[pl.ds(wbase, C)], i1s)
        gather_start(slot_a)
        idx_start(1, slot_b)

        def pair(k, carry):
            half(2 * k, slot_a, slot_b)
            half(2 * k + 1, slot_b, slot_a)
            return carry

        lax.fori_loop(0, nchunk // 2, pair, 0)
        out_wait(slot_a)
        out_wait(slot_b)

    return gk(var_p, con_p, idx0, idx1)


def _edge_body(ceT_ref, gT_ref, w1_ref, b1_ref, w2_ref, b2_ref, o_ref):
    pre = jnp.dot(w1_ref[...], ceT_ref[...],
                  preferred_element_type=jnp.float32) + gT_ref[...] + b1_ref[...]
    h = jnp.maximum(pre, 0.0)
    o_ref[...] = jnp.dot(w2_ref[...], h,
                         preferred_element_type=jnp.float32) + b2_ref[...]


def _edge(ceT, gT, w1e, b1, w2, b2, block=64000):
    block = min(block, E)
    wspec = pl.BlockSpec((16, 16), lambda i: (0, 0))
    bspec = pl.BlockSpec((16, 1), lambda i: (0, 0))
    return pl.pallas_call(
        _edge_body,
        grid=(E // block,),
        in_specs=[pl.BlockSpec((16, block), lambda i: (0, i)),
                  pl.BlockSpec((16, block), lambda i: (0, i)),
                  wspec, bspec, wspec, bspec],
        out_specs=pl.BlockSpec((16, block), lambda i: (0, i)),
        out_shape=jax.ShapeDtypeStruct((16, E), jnp.float32),
    )(ceT, gT, w1e.T, b1.reshape(16, 1), w2.T, b2.reshape(16, 1))


def kernel(var_f, con_f, combined_edge_f, edge_index_var_con,
           vW1, vb1, vW2, vb2, cW1, cb1, cW2, cb2, eW1, eb1, eW2, eb2):
    idx0 = edge_index_var_con[0]
    idx1 = edge_index_var_con[1]
    eW1_e, eW1_v, eW1_c = eW1[:16], eW1[16:32], eW1[32:48]
    var_pT = _node(var_f.T, vW1, vb1, vW2, vb2, eW1_v, block=N_VAR)
    con_pT = _node(con_f.T, cW1, cb1, cW2, cb2, eW1_c, block=N_CON)
    g8 = _gather_add(var_pT.T, con_pT.T, idx0, idx1)
    gT = g8.reshape(E, 16).T
    outT = _edge(combined_edge_f.T, gT, eW1_e, eb1, eW2, eb2)
    return outT.T


# speedup vs baseline: 8.1864x; 1.1794x over previous
import functools

import jax
import jax.numpy as jnp
from jax import lax
from jax.experimental import pallas as pl
from jax.experimental.pallas import tpu as pltpu
from jax.experimental.pallas import tpu_sc as plsc

N_VAR, N_CON, E, D = 100000, 50000, 1600000, 16


def _node_body(x_ref, w1_ref, b1_ref, w2_ref, b2_ref, wp_ref, o_ref):
    h = jnp.maximum(jnp.dot(w1_ref[...], x_ref[...],
                            preferred_element_type=jnp.float32) + b1_ref[...], 0.0)
    h = jnp.maximum(jnp.dot(w2_ref[...], h,
                            preferred_element_type=jnp.float32) + b2_ref[...], 0.0)
    o_ref[...] = jnp.dot(wp_ref[...], h, preferred_element_type=jnp.float32)


def _node(xT, w1, b1, w2, b2, wp, block):
    n = xT.shape[1]
    block = min(block, n)
    wspec = pl.BlockSpec((16, 16), lambda i: (0, 0))
    bspec = pl.BlockSpec((16, 1), lambda i: (0, 0))
    return pl.pallas_call(
        _node_body,
        grid=(n // block,),
        in_specs=[pl.BlockSpec((16, block), lambda i: (0, i)),
                  wspec, bspec, wspec, bspec, wspec],
        out_specs=pl.BlockSpec((16, block), lambda i: (0, i)),
        out_shape=jax.ShapeDtypeStruct((16, n), jnp.float32),
    )(xT, w1.T, b1.reshape(16, 1), w2.T, b2.reshape(16, 1), wp.T)


_CHUNK = 1000


def _gather_add(var_p, con_p, idx0, idx1, n, offset):
    info = plsc.get_sparse_core_info()
    nc, ns = info.num_cores, info.num_subcores
    nw = nc * ns
    epw = n // nw
    nchunk = epw // _CHUNK
    C = _CHUNK

    mesh = plsc.VectorSubcoreMesh(core_axis_name="c", subcore_axis_name="s")

    @functools.partial(
        pl.kernel,
        out_type=jax.ShapeDtypeStruct((n // 8, 128), jnp.float32),
        mesh=mesh,
        compiler_params=pltpu.CompilerParams(use_tc_tiling_on_sc=False),
        scratch_types=[
            pltpu.VMEM((C,), jnp.int32), pltpu.VMEM((C,), jnp.int32),
            pltpu.VMEM((C,), jnp.int32), pltpu.VMEM((C,), jnp.int32),
            pltpu.VMEM((C, 16), jnp.float32), pltpu.VMEM((C, 16), jnp.float32),
            pltpu.VMEM((C, 16), jnp.float32), pltpu.VMEM((C, 16), jnp.float32),
            pltpu.VMEM((C // 8, 128), jnp.float32), pltpu.VMEM((C // 8, 128), jnp.float32),
            pltpu.SemaphoreType.DMA, pltpu.SemaphoreType.DMA,
            pltpu.SemaphoreType.DMA, pltpu.SemaphoreType.DMA,
            pltpu.SemaphoreType.DMA, pltpu.SemaphoreType.DMA,
        ],
    )
    def gk(varp_hbm, conp_hbm, idx0_hbm, idx1_hbm, out_hbm,
           i0a, i0b, i1a, i1b, va, vb, ca, cb, oba, obb,
           gsa, gsb, isa, isb, osa, osb):
        wid = lax.axis_index("s") * nc + lax.axis_index("c")
        wbase = wid * epw
        ibase = offset + wbase
        slot_a = (i0a, i1a, va, ca, oba, gsa, isa, osa)
        slot_b = (i0b, i1b, vb, cb, obb, gsb, isb, osb)

        def idx_start(e, s):
            (i0s, i1s, _, _, _, _, iss, _) = s
            pltpu.async_copy(idx0_hbm.at[pl.ds(ibase + e * C, C)], i0s, iss)
            pltpu.async_copy(idx1_hbm.at[pl.ds(ibase + e * C, C)], i1s, iss)

        def idx_wait(s):
            (i0s, i1s, _, _, _, _, iss, _) = s
            pltpu.make_async_copy(idx0_hbm.at[pl.ds(ibase, C)], i0s, iss).wait()
            pltpu.make_async_copy(idx1_hbm.at[pl.ds(ibase, C)], i1s, iss).wait()

        def gather_start(s):
            (i0s, i1s, vs, cs, _, gss, _, _) = s
            pltpu.async_copy(varp_hbm.at[i0s], vs, gss)
            pltpu.async_copy(conp_hbm.at[i1s], cs, gss)

        def gather_wait(s):
            (i0s, i1s, vs, cs, _, gss, _, _) = s
            pltpu.make_async_copy(varp_hbm.at[i0s], vs, gss).wait()
            pltpu.make_async_copy(conp_hbm.at[i1s], cs, gss).wait()

        def out_wait(s):
            (_, _, _, _, obs, _, _, oss) = s
            pltpu.make_async_copy(
                obs, out_hbm.at[pl.ds(wbase // 8, C // 8), :], oss).wait()

        def half(e, s, n):
            (i0s, i1s, vs, cs, obs, gss, iss, oss) = s
            gather_wait(s)

            @pl.when(e + 1 < nchunk)
            def _():
                idx_wait(n)
                gather_start(n)

            @pl.when(e + 2 < nchunk)
            def _():
                idx_start(e + 2, s)

            @pl.when(e >= 2)
            def _():
                out_wait(s)

            def addrow(r, c2):
                obs[r // 8, pl.ds((r % 8) * 16, 16)] = vs[r, :] + cs[r, :]
                return c2
            lax.fori_loop(0, C, addrow, 0, unroll=8)
            pltpu.async_copy(
                obs, out_hbm.at[pl.ds((wbase + e * C) // 8, C // 8), :], oss)

        (i0s, i1s, _, _, _, _, _, _) = slot_a
        pltpu.sync_copy(idx0_hbm.at[pl.ds(ibase, C)], i0s)
        pltpu.sync_copy(idx1_hbm.at[pl.ds(ibase, C)], i1s)
        gather_start(slot_a)
        idx_start(1, slot_b)

        def pair(k, carry):
            half(2 * k, slot_a, slot_b)
            half(2 * k + 1, slot_b, slot_a)
            return carry

        lax.fori_loop(0, nchunk // 2, pair, 0)
        if nchunk % 2:
            half(nchunk - 1, slot_a, slot_b)
        out_wait(slot_a)
        out_wait(slot_b)

    return gk(var_p, con_p, idx0, idx1)


def _edge_body(ceT_ref, gT_ref, w1_ref, b1_ref, w2_ref, b2_ref, o_ref):
    pre = jnp.dot(w1_ref[...], ceT_ref[...],
                  preferred_element_type=jnp.float32) + gT_ref[...] + b1_ref[...]
    h = jnp.maximum(pre, 0.0)
    o_ref[...] = jnp.dot(w2_ref[...], h,
                         preferred_element_type=jnp.float32) + b2_ref[...]


def _edge_alias_body(ceT_ref, gT_ref, w1_ref, b1_ref, w2_ref, b2_ref,
                     prev_ref, o_ref):
    _edge_body(ceT_ref, gT_ref, w1_ref, b1_ref, w2_ref, b2_ref, o_ref)


def _edge_half(ceT, gTh, w1e, b1, w2, b2, h, prev, block=32000):
    nh = gTh.shape[1]
    block = min(block, nh)
    nblk = nh // block
    off = h * nblk
    wspec = pl.BlockSpec((16, 16), lambda i: (0, 0))
    bspec = pl.BlockSpec((16, 1), lambda i: (0, 0))
    in_specs = [pl.BlockSpec((16, block), lambda i: (0, i + off)),
                pl.BlockSpec((16, block), lambda i: (0, i)),
                wspec, bspec, wspec, bspec]
    args = [ceT, gTh, w1e.T, b1.reshape(16, 1), w2.T, b2.reshape(16, 1)]
    kwargs = {}
    body = _edge_body
    if prev is not None:
        in_specs.append(pl.BlockSpec(memory_space=pl.ANY))
        args.append(prev)
        kwargs["input_output_aliases"] = {6: 0}
        body = _edge_alias_body
    return pl.pallas_call(
        body,
        grid=(nblk,),
        in_specs=in_specs,
        out_specs=pl.BlockSpec((16, block), lambda i: (0, i + off)),
        out_shape=jax.ShapeDtypeStruct((16, E), jnp.float32),
        **kwargs,
    )(*args)


def kernel(var_f, con_f, combined_edge_f, edge_index_var_con,
           vW1, vb1, vW2, vb2, cW1, cb1, cW2, cb2, eW1, eb1, eW2, eb2):
    idx0 = edge_index_var_con[0]
    idx1 = edge_index_var_con[1]
    eW1_e, eW1_v, eW1_c = eW1[:16], eW1[16:32], eW1[32:48]
    var_pT = _node(var_f.T, vW1, vb1, vW2, vb2, eW1_v, block=N_VAR)
    con_pT = _node(con_f.T, cW1, cb1, cW2, cb2, eW1_c, block=N_CON)
    var_p, con_p = var_pT.T, con_pT.T
    ceT = combined_edge_f.T
    nh = E // 2
    out = None
    for h in range(2):
        g8 = _gather_add(var_p, con_p, idx0, idx1, nh, h * nh)
        gTh = g8.reshape(nh, 16).T
        out = _edge_half(ceT, gTh, eW1_e, eb1, eW2, eb2, h, out)
    return out.T


# speedup vs baseline: 8.6546x; 1.0572x over previous
import functools

import jax
import jax.numpy as jnp
from jax import lax
from jax.experimental import pallas as pl
from jax.experimental.pallas import tpu as pltpu
from jax.experimental.pallas import tpu_sc as plsc

N_VAR, N_CON, E, D = 100000, 50000, 1600000, 16


def _node_body(x_ref, w1_ref, b1_ref, w2_ref, b2_ref, wp_ref, o_ref):
    h = jnp.maximum(jnp.dot(w1_ref[...], x_ref[...],
                            preferred_element_type=jnp.float32) + b1_ref[...], 0.0)
    h = jnp.maximum(jnp.dot(w2_ref[...], h,
                            preferred_element_type=jnp.float32) + b2_ref[...], 0.0)
    o_ref[...] = jnp.dot(wp_ref[...], h, preferred_element_type=jnp.float32)


def _node(xT, w1, b1, w2, b2, wp, block):
    n = xT.shape[1]
    block = min(block, n)
    wspec = pl.BlockSpec((16, 16), lambda i: (0, 0))
    bspec = pl.BlockSpec((16, 1), lambda i: (0, 0))
    return pl.pallas_call(
        _node_body,
        grid=(n // block,),
        in_specs=[pl.BlockSpec((16, block), lambda i: (0, i)),
                  wspec, bspec, wspec, bspec, wspec],
        out_specs=pl.BlockSpec((16, block), lambda i: (0, i)),
        out_shape=jax.ShapeDtypeStruct((16, n), jnp.float32),
    )(xT, w1.T, b1.reshape(16, 1), w2.T, b2.reshape(16, 1), wp.T)


_CHUNK = 1000


def _gather_add(var_p, con_p, idx0, idx1, n, offset):
    info = plsc.get_sparse_core_info()
    nc, ns = info.num_cores, info.num_subcores
    nw = nc * ns
    epw = n // nw
    nchunk = epw // _CHUNK
    C = _CHUNK

    mesh = plsc.VectorSubcoreMesh(core_axis_name="c", subcore_axis_name="s")

    @functools.partial(
        pl.kernel,
        out_type=jax.ShapeDtypeStruct((n // 8, 128), jnp.float32),
        mesh=mesh,
        compiler_params=pltpu.CompilerParams(use_tc_tiling_on_sc=False),
        scratch_types=[
            pltpu.VMEM((C,), jnp.int32), pltpu.VMEM((C,), jnp.int32),
            pltpu.VMEM((C,), jnp.int32), pltpu.VMEM((C,), jnp.int32),
            pltpu.VMEM((C, 16), jnp.float32), pltpu.VMEM((C, 16), jnp.float32),
            pltpu.VMEM((C, 16), jnp.float32), pltpu.VMEM((C, 16), jnp.float32),
            pltpu.VMEM((C // 8, 128), jnp.float32), pltpu.VMEM((C // 8, 128), jnp.float32),
            pltpu.SemaphoreType.DMA, pltpu.SemaphoreType.DMA,
            pltpu.SemaphoreType.DMA, pltpu.SemaphoreType.DMA,
            pltpu.SemaphoreType.DMA, pltpu.SemaphoreType.DMA,
        ],
    )
    def gk(varp_hbm, conp_hbm, idx0_hbm, idx1_hbm, out_hbm,
           i0a, i0b, i1a, i1b, va, vb, ca, cb, oba, obb,
           gsa, gsb, isa, isb, osa, osb):
        wid = lax.axis_index("s") * nc + lax.axis_index("c")
        wbase = wid * epw
        ibase = offset + wbase
        slot_a = (i0a, i1a, va, ca, oba, gsa, isa, osa)
        slot_b = (i0b, i1b, vb, cb, obb, gsb, isb, osb)

        def idx_start(e, s):
            (i0s, i1s, _, _, _, _, iss, _) = s
            pltpu.async_copy(idx0_hbm.at[pl.ds(ibase + e * C, C)], i0s, iss)
            pltpu.async_copy(idx1_hbm.at[pl.ds(ibase + e * C, C)], i1s, iss)

        def idx_wait(s):
            (i0s, i1s, _, _, _, _, iss, _) = s
            pltpu.make_async_copy(idx0_hbm.at[pl.ds(ibase, C)], i0s, iss).wait()
            pltpu.make_async_copy(idx1_hbm.at[pl.ds(ibase, C)], i1s, iss).wait()

        def gather_start(s):
            (i0s, i1s, vs, cs, _, gss, _, _) = s
            pltpu.async_copy(varp_hbm.at[i0s], vs, gss)
            pltpu.async_copy(conp_hbm.at[i1s], cs, gss)

        def gather_wait(s):
            (i0s, i1s, vs, cs, _, gss, _, _) = s
            pltpu.make_async_copy(varp_hbm.at[i0s], vs, gss).wait()
            pltpu.make_async_copy(conp_hbm.at[i1s], cs, gss).wait()

        def out_wait(s):
            (_, _, _, _, obs, _, _, oss) = s
            pltpu.make_async_copy(
                obs, out_hbm.at[pl.ds(wbase // 8, C // 8), :], oss).wait()

        def half(e, s, n):
            (i0s, i1s, vs, cs, obs, gss, iss, oss) = s
            gather_wait(s)

            @pl.when(e + 1 < nchunk)
            def _():
                idx_wait(n)
                gather_start(n)

            @pl.when(e + 2 < nchunk)
            def _():
                idx_start(e + 2, s)

            @pl.when(e >= 2)
            def _():
                out_wait(s)

            def addrow(r, c2):
                obs[r // 8, pl.ds((r % 8) * 16, 16)] = vs[r, :] + cs[r, :]
                return c2
            lax.fori_loop(0, C, addrow, 0, unroll=8)
            pltpu.async_copy(
                obs, out_hbm.at[pl.ds((wbase + e * C) // 8, C // 8), :], oss)

        (i0s, i1s, _, _, _, _, _, _) = slot_a
        pltpu.sync_copy(idx0_hbm.at[pl.ds(ibase, C)], i0s)
        pltpu.sync_copy(idx1_hbm.at[pl.ds(ibase, C)], i1s)
        gather_start(slot_a)
        idx_start(1, slot_b)

        def pair(k, carry):
            half(2 * k, slot_a, slot_b)
            half(2 * k + 1, slot_b, slot_a)
            return carry

        lax.fori_loop(0, nchunk // 2, pair, 0)
        if nchunk % 2:
            half(nchunk - 1, slot_a, slot_b)
        out_wait(slot_a)
        out_wait(slot_b)

    return gk(var_p, con_p, idx0, idx1)


def _cast_body(x_ref, o_ref):
    o_ref[...] = x_ref[...].astype(jnp.bfloat16)


def _cast_bf16(x, block=8000):
    n = x.shape[0]
    block = min(block, n)
    return pl.pallas_call(
        _cast_body,
        grid=(n // block,),
        in_specs=[pl.BlockSpec((block, 128), lambda i: (i, 0))],
        out_specs=pl.BlockSpec((block, 128), lambda i: (i, 0)),
        out_shape=jax.ShapeDtypeStruct((n, 128), jnp.bfloat16),
    )(x)


def _edge_body(ceT_ref, gT_ref, w1_ref, b1_ref, w2_ref, b2_ref, o_ref):
    pre = jnp.dot(w1_ref[...], ceT_ref[...],
                  preferred_element_type=jnp.float32) \
        + gT_ref[...].astype(jnp.float32) + b1_ref[...]
    h = jnp.maximum(pre, 0.0)
    o_ref[...] = jnp.dot(w2_ref[...], h,
                         preferred_element_type=jnp.float32) + b2_ref[...]


def _edge_alias_body(ceT_ref, gT_ref, w1_ref, b1_ref, w2_ref, b2_ref,
                     prev_ref, o_ref):
    _edge_body(ceT_ref, gT_ref, w1_ref, b1_ref, w2_ref, b2_ref, o_ref)


def _edge_half(ceT, gTh, w1e, b1, w2, b2, h, prev, block=32000):
    nh = gTh.shape[1]
    block = min(block, nh)
    nblk = nh // block
    off = h * nblk
    wspec = pl.BlockSpec((16, 16), lambda i: (0, 0))
    bspec = pl.BlockSpec((16, 1), lambda i: (0, 0))
    in_specs = [pl.BlockSpec((16, block), lambda i: (0, i + off)),
                pl.BlockSpec((16, block), lambda i: (0, i)),
                wspec, bspec, wspec, bspec]
    args = [ceT, gTh, w1e.T, b1.reshape(16, 1), w2.T, b2.reshape(16, 1)]
    kwargs = {}
    body = _edge_body
    if prev is not None:
        in_specs.append(pl.BlockSpec(memory_space=pl.ANY))
        args.append(prev)
        kwargs["input_output_aliases"] = {6: 0}
        body = _edge_alias_body
    return pl.pallas_call(
        body,
        grid=(nblk,),
        in_specs=in_specs,
        out_specs=pl.BlockSpec((16, block), lambda i: (0, i + off)),
        out_shape=jax.ShapeDtypeStruct((16, E), jnp.float32),
        **kwargs,
    )(*args)


def kernel(var_f, con_f, combined_edge_f, edge_index_var_con,
           vW1, vb1, vW2, vb2, cW1, cb1, cW2, cb2, eW1, eb1, eW2, eb2):
    idx0 = edge_index_var_con[0]
    idx1 = edge_index_var_con[1]
    eW1_e, eW1_v, eW1_c = eW1[:16], eW1[16:32], eW1[32:48]
    var_pT = _node(var_f.T, vW1, vb1, vW2, vb2, eW1_v, block=N_VAR)
    con_pT = _node(con_f.T, cW1, cb1, cW2, cb2, eW1_c, block=N_CON)
    var_p, con_p = var_pT.T, con_pT.T
    ceT = combined_edge_f.T
    nh = E // 2
    out = None
    for h in range(2):
        g8 = _gather_add(var_p, con_p, idx0, idx1, nh, h * nh)
        gTh = _cast_bf16(g8).reshape(nh, 16).T
        out = _edge_half(ceT, gTh, eW1_e, eb1, eW2, eb2, h, out)
    return out.T


# speedup vs baseline: 8.9994x; 1.0398x over previous
import functools

import jax
import jax.numpy as jnp
from jax import lax
from jax.experimental import pallas as pl
from jax.experimental.pallas import tpu as pltpu
from jax.experimental.pallas import tpu_sc as plsc

N_VAR, N_CON, E, D = 100000, 50000, 1600000, 16


def _node_body(x_ref, w1_ref, b1_ref, w2_ref, b2_ref, wp_ref, o_ref):
    h = jnp.maximum(jnp.dot(w1_ref[...], x_ref[...],
                            preferred_element_type=jnp.float32) + b1_ref[...], 0.0)
    h = jnp.maximum(jnp.dot(w2_ref[...], h,
                            preferred_element_type=jnp.float32) + b2_ref[...], 0.0)
    o_ref[...] = jnp.dot(wp_ref[...], h, preferred_element_type=jnp.float32)


def _node(xT, w1, b1, w2, b2, wp, block):
    n = xT.shape[1]
    block = min(block, n)
    wspec = pl.BlockSpec((16, 16), lambda i: (0, 0))
    bspec = pl.BlockSpec((16, 1), lambda i: (0, 0))
    return pl.pallas_call(
        _node_body,
        grid=(n // block,),
        in_specs=[pl.BlockSpec((16, block), lambda i: (0, i)),
                  wspec, bspec, wspec, bspec, wspec],
        out_specs=pl.BlockSpec((16, block), lambda i: (0, i)),
        out_shape=jax.ShapeDtypeStruct((16, n), jnp.float32),
    )(xT, w1.T, b1.reshape(16, 1), w2.T, b2.reshape(16, 1), wp.T)


_CHUNK = 1000


def _gather_add(var_p, con_p, idx0, n, offset):
    info = plsc.get_sparse_core_info()
    nc, ns = info.num_cores, info.num_subcores
    nw = nc * ns
    epw = n // nw
    nchunk = epw // _CHUNK
    C = _CHUNK

    mesh = plsc.VectorSubcoreMesh(core_axis_name="c", subcore_axis_name="s")

    @functools.partial(
        pl.kernel,
        out_type=jax.ShapeDtypeStruct((n // 8, 128), jnp.float32),
        mesh=mesh,
        compiler_params=pltpu.CompilerParams(use_tc_tiling_on_sc=False),
        scratch_types=[
            pltpu.VMEM((C,), jnp.int32), pltpu.VMEM((C,), jnp.int32),
            pltpu.VMEM((C,), jnp.int32), pltpu.VMEM((C,), jnp.int32),
            pltpu.VMEM((C, 16), jnp.float32), pltpu.VMEM((C, 16), jnp.float32),
            pltpu.VMEM((C, 16), jnp.float32), pltpu.VMEM((C, 16), jnp.float32),
            pltpu.VMEM((C // 8, 128), jnp.float32), pltpu.VMEM((C // 8, 128), jnp.float32),
            pltpu.SemaphoreType.DMA, pltpu.SemaphoreType.DMA,
            pltpu.SemaphoreType.DMA, pltpu.SemaphoreType.DMA,
            pltpu.SemaphoreType.DMA, pltpu.SemaphoreType.DMA,
        ],
    )
    def gk(varp_hbm, conp_hbm, ei_hbm, out_hbm,
           i0a, i0b, i1a, i1b, va, vb, ca, cb, oba, obb,
           gsa, gsb, isa, isb, osa, osb):
        wid = lax.axis_index("s") * nc + lax.axis_index("c")
        wbase = wid * epw
        ibase = offset + wbase
        slot_a = (i0a, i1a, va, ca, oba, gsa, isa, osa)
        slot_b = (i0b, i1b, vb, cb, obb, gsb, isb, osb)

        def idx_start(e, s):
            (i0s, i1s, _, _, _, _, iss, _) = s
            pltpu.async_copy(ei_hbm.at[0, pl.ds(ibase + e * C, C)], i0s, iss)
            pltpu.async_copy(ei_hbm.at[1, pl.ds(ibase + e * C, C)], i1s, iss)

        def idx_wait(s):
            (i0s, i1s, _, _, _, _, iss, _) = s
            pltpu.make_async_copy(ei_hbm.at[0, pl.ds(ibase, C)], i0s, iss).wait()
            pltpu.make_async_copy(ei_hbm.at[1, pl.ds(ibase, C)], i1s, iss).wait()

        def gather_start(s):
            (i0s, i1s, vs, cs, _, gss, _, _) = s
            pltpu.async_copy(varp_hbm.at[i0s], vs, gss)
            pltpu.async_copy(conp_hbm.at[i1s], cs, gss)

        def gather_wait(s):
            (i0s, i1s, vs, cs, _, gss, _, _) = s
            pltpu.make_async_copy(varp_hbm.at[i0s], vs, gss).wait()
            pltpu.make_async_copy(conp_hbm.at[i1s], cs, gss).wait()

        def out_wait(s):
            (_, _, _, _, obs, _, _, oss) = s
            pltpu.make_async_copy(
                obs, out_hbm.at[pl.ds(wbase // 8, C // 8), :], oss).wait()

        def half(e, s, n):
            (i0s, i1s, vs, cs, obs, gss, iss, oss) = s
            gather_wait(s)

            @pl.when(e + 1 < nchunk)
            def _():
                idx_wait(n)
                gather_start(n)

            @pl.when(e + 2 < nchunk)
            def _():
                idx_start(e + 2, s)

            @pl.when(e >= 2)
            def _():
                out_wait(s)

            def addrow(r, c2):
                obs[r // 8, pl.ds((r % 8) * 16, 16)] = vs[r, :] + cs[r, :]
                return c2
            lax.fori_loop(0, C, addrow, 0, unroll=8)
            pltpu.async_copy(
                obs, out_hbm.at[pl.ds((wbase + e * C) // 8, C // 8), :], oss)

        (i0s, i1s, _, _, _, _, _, _) = slot_a
        pltpu.sync_copy(ei_hbm.at[0, pl.ds(ibase, C)], i0s)
        pltpu.sync_copy(ei_hbm.at[1, pl.ds(ibase, C)], i1s)
        gather_start(slot_a)
        idx_start(1, slot_b)

        def pair(k, carry):
            half(2 * k, slot_a, slot_b)
            half(2 * k + 1, slot_b, slot_a)
            return carry

        lax.fori_loop(0, nchunk // 2, pair, 0)
        if nchunk % 2:
            half(nchunk - 1, slot_a, slot_b)
        out_wait(slot_a)
        out_wait(slot_b)

    return gk(var_p, con_p, idx0)


def _cast_body(x_ref, o_ref):
    o_ref[...] = x_ref[...].astype(jnp.bfloat16)


def _cast_bf16(x, block=8000):
    n = x.shape[0]
    block = min(block, n)
    return pl.pallas_call(
        _cast_body,
        grid=(n // block,),
        in_specs=[pl.BlockSpec((block, 128), lambda i: (i, 0))],
        out_specs=pl.BlockSpec((block, 128), lambda i: (i, 0)),
        out_shape=jax.ShapeDtypeStruct((n, 128), jnp.bfloat16),
    )(x)


def _edge_body(ceT_ref, gT_ref, w1_ref, b1_ref, w2_ref, b2_ref, o_ref):
    pre = jnp.dot(w1_ref[...], ceT_ref[...],
                  preferred_element_type=jnp.float32) \
        + gT_ref[...].astype(jnp.float32) + b1_ref[...]
    h = jnp.maximum(pre, 0.0)
    o_ref[...] = jnp.dot(w2_ref[...], h,
                         preferred_element_type=jnp.float32) + b2_ref[...]


def _edge_alias_body(ceT_ref, gT_ref, w1_ref, b1_ref, w2_ref, b2_ref,
                     prev_ref, o_ref):
    _edge_body(ceT_ref, gT_ref, w1_ref, b1_ref, w2_ref, b2_ref, o_ref)


def _edge_half(ceT, gTh, w1e, b1, w2, b2, h, prev, block=32000):
    nh = gTh.shape[1]
    block = min(block, nh)
    nblk = nh // block
    off = h * nblk
    wspec = pl.BlockSpec((16, 16), lambda i: (0, 0))
    bspec = pl.BlockSpec((16, 1), lambda i: (0, 0))
    in_specs = [pl.BlockSpec((16, block), lambda i: (0, i + off)),
                pl.BlockSpec((16, block), lambda i: (0, i)),
                wspec, bspec, wspec, bspec]
    args = [ceT, gTh, w1e.T, b1.reshape(16, 1), w2.T, b2.reshape(16, 1)]
    kwargs = {}
    body = _edge_body
    if prev is not None:
        in_specs.append(pl.BlockSpec(memory_space=pl.ANY))
        args.append(prev)
        kwargs["input_output_aliases"] = {6: 0}
        body = _edge_alias_body
    return pl.pallas_call(
        body,
        grid=(nblk,),
        in_specs=in_specs,
        out_specs=pl.BlockSpec((16, block), lambda i: (0, i + off)),
        out_shape=jax.ShapeDtypeStruct((16, E), jnp.float32),
        **kwargs,
    )(*args)


def kernel(var_f, con_f, combined_edge_f, edge_index_var_con,
           vW1, vb1, vW2, vb2, cW1, cb1, cW2, cb2, eW1, eb1, eW2, eb2):
    eW1_e, eW1_v, eW1_c = eW1[:16], eW1[16:32], eW1[32:48]
    var_pT = _node(var_f.T, vW1, vb1, vW2, vb2, eW1_v, block=N_VAR)
    con_pT = _node(con_f.T, cW1, cb1, cW2, cb2, eW1_c, block=N_CON)
    var_p, con_p = var_pT.T, con_pT.T
    ceT = combined_edge_f.T
    nh = E // 2
    out = None
    for h in range(2):
        g8 = _gather_add(var_p, con_p, edge_index_var_con, nh, h * nh)
        gTh = _cast_bf16(g8).reshape(nh, 16).T
        out = _edge_half(ceT, gTh, eW1_e, eb1, eW2, eb2, h, out)
    return out.T
